# Initial kernel scaffold; baseline (speedup 1.0000x reference)
import jax
import jax.numpy as jnp
from jax.experimental import pallas as pl


def kernel(vertices, d0, w1, b1, d1, w2, b2, d2, w3, b3, d3, w4, b4, d4, Wa, ba, Wb, bb, Wc, bc):
    raise NotImplementedError("write your pallas kernel here")


# speedup vs baseline: 9.3425x; 9.3425x over previous
import functools

import numpy as np
import jax
import jax.numpy as jnp
from jax import lax
from jax.experimental import pallas as pl
from jax.experimental.pallas import tpu as pltpu
from jax.experimental.pallas import tpu_sc as plsc

_S1 = np.random.RandomState(0).permutation(2048)[:512]
_S2 = np.random.RandomState(1).permutation(512)[:128]

_NW = 32


def _bdot(a, b):
    return jnp.dot(a.astype(jnp.bfloat16), b.astype(jnp.bfloat16),
                   preferred_element_type=jnp.float32)


@functools.lru_cache(maxsize=None)
def _make_sc_gather(T, D, B):
    n = B // _NW
    C = min(128, n)
    nchunk = n // C
    assert n % C == 0 and B % _NW == 0
    mesh = plsc.VectorSubcoreMesh(
        core_axis_name="c", subcore_axis_name="s", num_cores=2,
        num_subcores=16)

    @functools.partial(
        pl.kernel,
        out_type=jax.ShapeDtypeStruct((B, D), jnp.float32),
        mesh=mesh,
        scratch_types=[
            pltpu.VMEM((nchunk, C), jnp.int32),
            pltpu.VMEM((C, D), jnp.float32),
            pltpu.SemaphoreType.DMA,
        ],
    )
    def gather(table_hbm, idx_hbm, out_hbm, idx_v, rows_v, sem):
        wid = lax.axis_index("s") * 2 + lax.axis_index("c")
        row0 = wid * nchunk
        pltpu.sync_copy(idx_hbm.at[pl.ds(row0, nchunk)], idx_v)
        for j in range(nchunk):
            pltpu.async_copy(table_hbm.at[idx_v.at[j]], rows_v, sem).wait()
            pltpu.sync_copy(rows_v, out_hbm.at[pl.ds((row0 + j) * C, C)])

    return gather


def _sc_gather(table, idx):
    T, D = table.shape
    B = idx.shape[0]
    n = B // _NW
    C = min(128, n)
    idx2d = idx.reshape(B // C, C)
    return _make_sc_gather(T, D, B)(table, idx2d)


def _knn_body(R, V, K, vb_ref, vT_ref, ni_ref, dist_ref):
    vb = vb_ref[0]
    vT = vT_ref[0]
    g = _bdot(vb, vT)
    q_all = jnp.sum(vT * vT, axis=0, keepdims=True)
    q_row = jnp.sum(vb * vb, axis=1, keepdims=True)
    col = lax.broadcasted_iota(jnp.int32, (R, V), 1)
    dist_ref[...] = (-2.0 * g + q_all) + q_row
    kiota = lax.broadcasted_iota(jnp.int32, (R, K), 1)

    def step(j, ids):
        d = dist_ref[...]
        m = jnp.min(d, axis=1, keepdims=True)
        idx = jnp.min(jnp.where(d == m, col, V), axis=1, keepdims=True)
        dist_ref[...] = jnp.where(col == idx, jnp.inf, d)
        return jnp.where(kiota == j - 1, idx, ids)

    ni_ref[0] = lax.fori_loop(0, K + 1, step, jnp.zeros((R, K), jnp.int32))


def _knn(v, vT, R, K=32):
    bs, V, _ = v.shape
    grid = (bs, V // R)
    return pl.pallas_call(
        functools.partial(_knn_body, R, V, K),
        grid=grid,
        in_specs=[
            pl.BlockSpec((1, R, 3), lambda b, r: (b, r, 0)),
            pl.BlockSpec((1, 3, V), lambda b, r: (b, 0, 0)),
        ],
        out_specs=pl.BlockSpec((1, R, K), lambda b, r: (b, r, 0)),
        out_shape=jax.ShapeDtypeStruct((bs, V, K), jnp.int32),
        scratch_shapes=[pltpu.VMEM((R, V), jnp.float32)],
    )(v, vT)


def _theta(gv, vb, d, R, N):
    n = jnp.sqrt(jnp.sum(d * d, axis=0, keepdims=True))
    sd = d / jnp.maximum(n, 1e-12)
    diff = gv[:, :, :3] - vb[:, None, :]
    nrm = jnp.sqrt(jnp.sum(diff * diff, axis=2, keepdims=True))
    nd = diff / jnp.maximum(nrm, 1e-12)
    C = d.shape[1]
    return jnp.maximum(_bdot(nd.reshape(R * N, 3), sd), 0.0) \
        .reshape(R, N, C)


def _kA_body(R, gv_ref, v_ref, d_ref, w_ref, b_ref, fm0_ref, c1_ref, s1_ref):
    th = _theta(gv_ref[0], v_ref[0], d_ref[...], R, 32)
    fm0 = jnp.maximum(jnp.max(th, axis=1), 0.0)
    fout = _bdot(fm0, w_ref[...]) + b_ref[...]
    fm0_ref[0] = fm0
    c1_ref[0] = fout[:, :128]
    s1_ref[0] = fout[:, 128:]


def _comb_body(R, relu, w_next, *refs):
    if w_next:
        (gv_ref, v_ref, d_ref, c_ref, sup_ref, w_ref, b_ref,
         fm_ref, cn_ref, sn_ref) = refs
    else:
        gv_ref, v_ref, d_ref, c_ref, sup_ref, fm_ref = refs
    th = _theta(gv_ref[0], v_ref[0], d_ref[...], R, 32)
    out = c_ref[0] + jnp.max(th * sup_ref[0], axis=1)
    if relu:
        out = jnp.maximum(out, 0.0)
    fm_ref[0] = out
    if w_next:
        fout = _bdot(out, w_ref[...]) + b_ref[...]
        half = w_ref.shape[1] // 2
        cn_ref[0] = fout[:, :half]
        sn_ref[0] = fout[:, half:]


def _comb(GV, v, d, c, sup, R, relu=True, w=None, b=None):
    bs, V, N, C = sup.shape
    grid = (bs, V // R)
    in_arrs = [GV, v, d, c, sup]
    in_specs = [
        pl.BlockSpec((1, R, N, 128), lambda bi, r: (bi, r, 0, 0)),
        pl.BlockSpec((1, R, 3), lambda bi, r: (bi, r, 0)),
        pl.BlockSpec(d.shape, lambda bi, r: (0, 0)),
        pl.BlockSpec((1, R, C), lambda bi, r: (bi, r, 0)),
        pl.BlockSpec((1, R, N, C), lambda bi, r: (bi, r, 0, 0)),
    ]
    out_shapes = [jax.ShapeDtypeStruct((bs, V, C), jnp.float32)]
    out_specs = [pl.BlockSpec((1, R, C), lambda bi, r: (bi, r, 0))]
    if w is not None:
        in_arrs += [w, b]
        in_specs += [pl.BlockSpec(w.shape, lambda bi, r: (0, 0)),
                     pl.BlockSpec(b.shape, lambda bi, r: (0, 0))]
        half = w.shape[1] // 2
        out_shapes += [jax.ShapeDtypeStruct((bs, V, half), jnp.float32)] * 2
        out_specs += [pl.BlockSpec((1, R, half), lambda bi, r: (bi, r, 0))] * 2
    return pl.pallas_call(
        functools.partial(_comb_body, R, relu, w is not None),
        grid=grid, in_specs=in_specs, out_specs=out_specs,
        out_shape=out_shapes,
    )(*in_arrs)


def _pool_mm_body(g_ref, w_ref, b_ref, c_ref, s_ref):
    fmp = jnp.max(g_ref[0], axis=1)
    fout = _bdot(fmp, w_ref[...]) + b_ref[...]
    half = w_ref.shape[1] // 2
    c_ref[0] = fout[:, :half]
    s_ref[0] = fout[:, half:]


def _pool_mm(gp, w, b):
    bs, V, pn, Cin = gp.shape
    half = w.shape[1] // 2
    return pl.pallas_call(
        _pool_mm_body,
        grid=(bs,),
        in_specs=[
            pl.BlockSpec((1, V, pn, Cin), lambda bi: (bi, 0, 0, 0)),
            pl.BlockSpec(w.shape, lambda bi: (0, 0)),
            pl.BlockSpec(b.shape, lambda bi: (0, 0)),
        ],
        out_specs=[pl.BlockSpec((1, V, half), lambda bi: (bi, 0, 0))] * 2,
        out_shape=[jax.ShapeDtypeStruct((bs, V, half), jnp.float32)] * 2,
    )(gp, w, b)


def _kC2_body(gv_ref, v_ref, d_ref, c_ref, sup_ref, fm_ref, gl_ref):
    th = _theta(gv_ref[0], v_ref[0], d_ref[...], 128, 32)
    fm4 = c_ref[0] + jnp.max(th * sup_ref[0], axis=1)
    fm_ref[0] = fm4
    gl_ref[0] = jnp.max(fm4, axis=0, keepdims=True)


def _near_body(v_ref, p1_ref, p2_ref, n1_ref, n2_ref):
    vb = v_ref[0]
    t2 = jnp.sum(vb * vb, axis=1, keepdims=True)

    def nearest(pT):
        g = _bdot(vb, pT)
        s2 = jnp.sum(pT * pT, axis=0, keepdims=True)
        dist = (s2 + t2) - 2.0 * g
        m = jnp.min(dist, axis=1, keepdims=True)
        col = lax.broadcasted_iota(jnp.int32, dist.shape, 1)
        W = dist.shape[1]
        return jnp.min(jnp.where(dist == m, col, W), axis=1)

    n1_ref[0, 0] = nearest(p1_ref[0])
    n2_ref[0, 0] = nearest(p2_ref[0])


def _head_body(fm0_ref, fm1_ref, u23_ref, u4_ref, gl_ref,
               wa0, wa1, wa23, wa4, wag, ba, wb, bb, wc, bc, out_ref):
    x = (_bdot(fm0_ref[0], wa0[...]) + _bdot(fm1_ref[0], wa1[...])
         + _bdot(u23_ref[0], wa23[...]) + _bdot(u4_ref[0], wa4[...])
         + _bdot(gl_ref[0], wag[...]) + ba[...])
    x = jnp.maximum(x, 0.0)
    x = jnp.maximum(_bdot(x, wb[...]) + bb[...], 0.0)
    out_ref[0] = _bdot(x, wc[...]) + bc[...]


def kernel(vertices, d0, w1, b1, d1, w2, b2, d2, w3, b3, d3, w4, b4, d4,
           Wa, ba, Wb, bb, Wc, bc):
    bs, V, _ = vertices.shape
    f32 = jnp.float32
    ba2, bb2, bc2 = ba[None], bb[None], bc[None]
    b1_2, b2_2, b3_2, b4_2 = b1[None], b2[None], b3[None], b4[None]

    vT = jnp.transpose(vertices, (0, 2, 1))
    ni = _knn(vertices, vT, R=512)

    off0 = (jnp.arange(bs, dtype=jnp.int32) * V)[:, None, None]
    idxA = (ni + off0).reshape(-1)

    cpad0 = jnp.pad(vertices, ((0, 0), (0, 0), (0, 125)))
    GV = _sc_gather(cpad0.reshape(bs * V, 128), idxA).reshape(bs, V, 32, 128)

    fm0, c1, s1 = pl.pallas_call(
        functools.partial(_kA_body, 256),
        grid=(bs, V // 256),
        in_specs=[
            pl.BlockSpec((1, 256, 32, 128), lambda b, r: (b, r, 0, 0)),
            pl.BlockSpec((1, 256, 3), lambda b, r: (b, r, 0)),
            pl.BlockSpec((3, 128), lambda b, r: (0, 0)),
            pl.BlockSpec((128, 256), lambda b, r: (0, 0)),
            pl.BlockSpec((1, 256), lambda b, r: (0, 0)),
        ],
        out_specs=[pl.BlockSpec((1, 256, 128), lambda b, r: (b, r, 0))] * 3,
        out_shape=[jax.ShapeDtypeStruct((bs, V, 128), f32)] * 3,
    )(GV, vertices, d0, w1, b1_2)

    g1 = _sc_gather(s1.reshape(bs * V, 128), idxA).reshape(bs, V, 32, 128)
    (fm1,) = _comb(GV, vertices, d1, c1, g1, R=256)

    ip1 = (ni[:, _S1, :4] + off0).reshape(-1)
    gp1 = _sc_gather(fm1.reshape(bs * V, 128), ip1).reshape(bs, 512, 4, 128)
    vp1 = vertices[:, _S1, :]
    vp1T = jnp.transpose(vp1, (0, 2, 1))

    ni2 = _knn(vp1, vp1T, R=512)
    off1 = (jnp.arange(bs, dtype=jnp.int32) * 512)[:, None, None]
    idxB = (ni2 + off1).reshape(-1)

    cpad1 = jnp.pad(vp1, ((0, 0), (0, 0), (0, 125)))
    GV2 = _sc_gather(cpad1.reshape(bs * 512, 128),
                     idxB).reshape(bs, 512, 32, 128)

    c2, s2 = _pool_mm(gp1, w2, b2_2)
    g2 = _sc_gather(s2.reshape(bs * 512, 256), idxB).reshape(bs, 512, 32, 256)
    fm2, c3, s3 = _comb(GV2, vp1, d2, c2, g2, R=128, w=w3, b=b3_2)
    g3 = _sc_gather(s3.reshape(bs * 512, 256), idxB).reshape(bs, 512, 32, 256)
    (fm3,) = _comb(GV2, vp1, d3, c3, g3, R=128)

    ip2 = (ni2[:, _S2, :4] + off1).reshape(-1)
    gp2 = _sc_gather(fm3.reshape(bs * 512, 256), ip2).reshape(bs, 128, 4, 256)
    vp2 = vp1[:, _S2, :]
    vp2T = jnp.transpose(vp2, (0, 2, 1))

    ni3 = _knn(vp2, vp2T, R=128)
    off2 = (jnp.arange(bs, dtype=jnp.int32) * 128)[:, None, None]
    idxC = (ni3 + off2).reshape(-1)

    cpad2 = jnp.pad(vp2, ((0, 0), (0, 0), (0, 125)))
    GV3 = _sc_gather(cpad2.reshape(bs * 128, 128),
                     idxC).reshape(bs, 128, 32, 128)

    c4, s4 = _pool_mm(gp2, w4, b4_2)
    g4 = _sc_gather(s4.reshape(bs * 128, 512), idxC).reshape(bs, 128, 32, 512)

    fm4, fglob = pl.pallas_call(
        _kC2_body,
        grid=(bs,),
        in_specs=[
            pl.BlockSpec((1, 128, 32, 128), lambda b: (b, 0, 0, 0)),
            pl.BlockSpec((1, 128, 3), lambda b: (b, 0, 0)),
            pl.BlockSpec((3, 512), lambda b: (0, 0)),
            pl.BlockSpec((1, 128, 512), lambda b: (b, 0, 0)),
            pl.BlockSpec((1, 128, 32, 512), lambda b: (b, 0, 0, 0)),
        ],
        out_specs=[pl.BlockSpec((1, 128, 512), lambda b: (b, 0, 0)),
                   pl.BlockSpec((1, 1, 512), lambda b: (b, 0, 0))],
        out_shape=[jax.ShapeDtypeStruct((bs, 128, 512), f32),
                   jax.ShapeDtypeStruct((bs, 1, 512), f32)],
    )(GV3, vp2, d4, c4, g4)

    np1, np2 = pl.pallas_call(
        _near_body,
        grid=(bs, V // 512),
        in_specs=[
            pl.BlockSpec((1, 512, 3), lambda b, r: (b, r, 0)),
            pl.BlockSpec((1, 3, 512), lambda b, r: (b, 0, 0)),
            pl.BlockSpec((1, 3, 128), lambda b, r: (b, 0, 0)),
        ],
        out_specs=[pl.BlockSpec((1, 1, 512),
                                lambda b, r: (b * (V // 512) + r, 0, 0))] * 2,
        out_shape=[jax.ShapeDtypeStruct((bs * (V // 512), 1, 512),
                                        jnp.int32)] * 2,
    )(vertices, vp1T, vp2T)
    np1 = np1.reshape(bs, V)
    np2 = np2.reshape(bs, V)

    fm23 = jnp.concatenate([fm2, fm3], axis=-1)
    u23 = _sc_gather(fm23.reshape(bs * 512, 512),
                     (np1 + off1[:, :, 0]).reshape(-1)).reshape(bs, V, 512)
    u4 = _sc_gather(fm4.reshape(bs * 128, 512),
                    (np2 + off2[:, :, 0]).reshape(-1)).reshape(bs, V, 512)

    out = pl.pallas_call(
        _head_body,
        grid=(bs, V // 512),
        in_specs=[
            pl.BlockSpec((1, 512, 128), lambda b, r: (b, r, 0)),
            pl.BlockSpec((1, 512, 128), lambda b, r: (b, r, 0)),
            pl.BlockSpec((1, 512, 512), lambda b, r: (b, r, 0)),
            pl.BlockSpec((1, 512, 512), lambda b, r: (b, r, 0)),
            pl.BlockSpec((1, 1, 512), lambda b, r: (b, 0, 0)),
            pl.BlockSpec((128, 512), lambda b, r: (0, 0)),
            pl.BlockSpec((128, 512), lambda b, r: (0, 0)),
            pl.BlockSpec((512, 512), lambda b, r: (0, 0)),
            pl.BlockSpec((512, 512), lambda b, r: (0, 0)),
            pl.BlockSpec((512, 512), lambda b, r: (0, 0)),
            pl.BlockSpec((1, 512), lambda b, r: (0, 0)),
            pl.BlockSpec((512, 512), lambda b, r: (0, 0)),
            pl.BlockSpec((1, 512), lambda b, r: (0, 0)),
            pl.BlockSpec((512, 50), lambda b, r: (0, 0)),
            pl.BlockSpec((1, 50), lambda b, r: (0, 0)),
        ],
        out_specs=pl.BlockSpec((1, 512, 50), lambda b, r: (b, r, 0)),
        out_shape=jax.ShapeDtypeStruct((bs, V, 50), f32),
    )(fm0, fm1, u23, u4, fglob, Wa[:128], Wa[128:256], Wa[256:768],
      Wa[768:1280], Wa[1280:1792], ba2, Wb, bb2, Wc, bc2)

    return out


# speedup vs baseline: 9.6505x; 1.0330x over previous
import functools

import numpy as np
import jax
import jax.numpy as jnp
from jax import lax
from jax.experimental import pallas as pl
from jax.experimental.pallas import tpu as pltpu
from jax.experimental.pallas import tpu_sc as plsc

_S1 = np.random.RandomState(0).permutation(2048)[:512]
_S2 = np.random.RandomState(1).permutation(512)[:128]

_NW = 32


def _bdot(a, b):
    return jnp.dot(a.astype(jnp.bfloat16), b.astype(jnp.bfloat16),
                   preferred_element_type=jnp.float32)


def _chunk_len(n, D):
    cap = min(128, n, 32768 // D)
    C = 1
    for c in range(1, cap + 1):
        if n % c == 0:
            C = c
    return C


@functools.lru_cache(maxsize=None)
def _make_sc_gather(T, D, B):
    n = B // _NW
    C = _chunk_len(n, D)
    nchunk = n // C
    mesh = plsc.VectorSubcoreMesh(
        core_axis_name="c", subcore_axis_name="s", num_cores=2,
        num_subcores=16)

    @functools.partial(
        pl.kernel,
        out_type=jax.ShapeDtypeStruct((B, D), jnp.float32),
        mesh=mesh,
        scratch_types=[
            pltpu.VMEM((nchunk, C), jnp.int32),
            pltpu.VMEM((C, D), jnp.float32),
            pltpu.VMEM((C, D), jnp.float32),
            pltpu.SemaphoreType.DMA,
            pltpu.SemaphoreType.DMA,
            pltpu.SemaphoreType.DMA,
            pltpu.SemaphoreType.DMA,
        ],
    )
    def gather(table_hbm, idx_hbm, out_hbm, idx_v, rows_a, rows_b,
               gs_a, gs_b, ws_a, ws_b):
        wid = lax.axis_index("s") * 2 + lax.axis_index("c")
        row0 = wid * nchunk
        pltpu.sync_copy(idx_hbm.at[pl.ds(row0, nchunk)], idx_v)
        rows = (rows_a, rows_b)
        gsem = (gs_a, gs_b)
        wsem = (ws_a, ws_b)
        gh = [None, None]
        wh = [None, None]
        for j in range(nchunk):
            b = j & 1
            if wh[b] is not None:
                wh[b].wait()
            gh[b] = pltpu.async_copy(table_hbm.at[idx_v.at[j]], rows[b],
                                     gsem[b])
            if j >= 1:
                pb = (j - 1) & 1
                gh[pb].wait()
                wh[pb] = pltpu.async_copy(
                    rows[pb], out_hbm.at[pl.ds((row0 + j - 1) * C, C)],
                    wsem[pb])
        lb = (nchunk - 1) & 1
        gh[lb].wait()
        pltpu.async_copy(rows[lb],
                         out_hbm.at[pl.ds((row0 + nchunk - 1) * C, C)],
                         wsem[lb]).wait()
        if nchunk >= 2 and wh[1 - lb] is not None:
            wh[1 - lb].wait()

    return gather


def _sc_gather(table, idx):
    T, D = table.shape
    B = idx.shape[0]
    n = B // _NW
    C = _chunk_len(n, D)
    idx2d = idx.reshape(B // C, C)
    return _make_sc_gather(T, D, B)(table, idx2d)


def _knn_body(R, V, K, vb_ref, vT_ref, ni_ref, dist_ref):
    vb = vb_ref[0]
    vT = vT_ref[0]
    g = _bdot(vb, vT)
    q_all = jnp.sum(vT * vT, axis=0, keepdims=True)
    q_row = jnp.sum(vb * vb, axis=1, keepdims=True)
    col = lax.broadcasted_iota(jnp.int32, (R, V), 1)
    dist_ref[...] = (-2.0 * g + q_all) + q_row
    kiota = lax.broadcasted_iota(jnp.int32, (R, K), 1)

    def step(j, ids):
        d = dist_ref[...]
        m = jnp.min(d, axis=1, keepdims=True)
        idx = jnp.min(jnp.where(d == m, col, V), axis=1, keepdims=True)
        dist_ref[...] = jnp.where(col == idx, jnp.inf, d)
        return jnp.where(kiota == j - 1, idx, ids)

    ni_ref[0] = lax.fori_loop(0, K + 1, step, jnp.zeros((R, K), jnp.int32))


def _knn(v, vT, R, K=32):
    bs, V, _ = v.shape
    grid = (bs, V // R)
    return pl.pallas_call(
        functools.partial(_knn_body, R, V, K),
        grid=grid,
        in_specs=[
            pl.BlockSpec((1, R, 3), lambda b, r: (b, r, 0)),
            pl.BlockSpec((1, 3, V), lambda b, r: (b, 0, 0)),
        ],
        out_specs=pl.BlockSpec((1, R, K), lambda b, r: (b, r, 0)),
        out_shape=jax.ShapeDtypeStruct((bs, V, K), jnp.int32),
        scratch_shapes=[pltpu.VMEM((R, V), jnp.float32)],
    )(v, vT)


def _theta(gv, vb, d, R, N):
    n = jnp.sqrt(jnp.sum(d * d, axis=0, keepdims=True))
    sd = d / jnp.maximum(n, 1e-12)
    diff = gv[:, :, :3] - vb[:, None, :]
    nrm = jnp.sqrt(jnp.sum(diff * diff, axis=2, keepdims=True))
    nd = diff / jnp.maximum(nrm, 1e-12)
    C = d.shape[1]
    return jnp.maximum(_bdot(nd.reshape(R * N, 3), sd), 0.0) \
        .reshape(R, N, C)


def _kA_body(R, gv_ref, v_ref, d_ref, w_ref, b_ref, fm0_ref, c1_ref, s1_ref):
    th = _theta(gv_ref[0], v_ref[0], d_ref[...], R, 32)
    fm0 = jnp.maximum(jnp.max(th, axis=1), 0.0)
    fout = _bdot(fm0, w_ref[...]) + b_ref[...]
    fm0_ref[0] = fm0
    c1_ref[0] = fout[:, :128]
    s1_ref[0] = fout[:, 128:]


def _comb_body(R, relu, w_next, *refs):
    if w_next:
        (gv_ref, v_ref, d_ref, c_ref, sup_ref, w_ref, b_ref,
         fm_ref, cn_ref, sn_ref) = refs
    else:
        gv_ref, v_ref, d_ref, c_ref, sup_ref, fm_ref = refs
    th = _theta(gv_ref[0], v_ref[0], d_ref[...], R, 32)
    out = c_ref[0] + jnp.max(th * sup_ref[0], axis=1)
    if relu:
        out = jnp.maximum(out, 0.0)
    fm_ref[0] = out
    if w_next:
        fout = _bdot(out, w_ref[...]) + b_ref[...]
        half = w_ref.shape[1] // 2
        cn_ref[0] = fout[:, :half]
        sn_ref[0] = fout[:, half:]


def _comb(GV, v, d, c, sup, R, relu=True, w=None, b=None):
    bs, V, N, C = sup.shape
    grid = (bs, V // R)
    in_arrs = [GV, v, d, c, sup]
    in_specs = [
        pl.BlockSpec((1, R, N, 128), lambda bi, r: (bi, r, 0, 0)),
        pl.BlockSpec((1, R, 3), lambda bi, r: (bi, r, 0)),
        pl.BlockSpec(d.shape, lambda bi, r: (0, 0)),
        pl.BlockSpec((1, R, C), lambda bi, r: (bi, r, 0)),
        pl.BlockSpec((1, R, N, C), lambda bi, r: (bi, r, 0, 0)),
    ]
    out_shapes = [jax.ShapeDtypeStruct((bs, V, C), jnp.float32)]
    out_specs = [pl.BlockSpec((1, R, C), lambda bi, r: (bi, r, 0))]
    if w is not None:
        in_arrs += [w, b]
        in_specs += [pl.BlockSpec(w.shape, lambda bi, r: (0, 0)),
                     pl.BlockSpec(b.shape, lambda bi, r: (0, 0))]
        half = w.shape[1] // 2
        out_shapes += [jax.ShapeDtypeStruct((bs, V, half), jnp.float32)] * 2
        out_specs += [pl.BlockSpec((1, R, half), lambda bi, r: (bi, r, 0))] * 2
    return pl.pallas_call(
        functools.partial(_comb_body, R, relu, w is not None),
        grid=grid, in_specs=in_specs, out_specs=out_specs,
        out_shape=out_shapes,
    )(*in_arrs)


def _pool_mm_body(g_ref, w_ref, b_ref, c_ref, s_ref):
    fmp = jnp.max(g_ref[0], axis=1)
    fout = _bdot(fmp, w_ref[...]) + b_ref[...]
    half = w_ref.shape[1] // 2
    c_ref[0] = fout[:, :half]
    s_ref[0] = fout[:, half:]


def _pool_mm(gp, w, b):
    bs, V, pn, Cin = gp.shape
    half = w.shape[1] // 2
    return pl.pallas_call(
        _pool_mm_body,
        grid=(bs,),
        in_specs=[
            pl.BlockSpec((1, V, pn, Cin), lambda bi: (bi, 0, 0, 0)),
            pl.BlockSpec(w.shape, lambda bi: (0, 0)),
            pl.BlockSpec(b.shape, lambda bi: (0, 0)),
        ],
        out_specs=[pl.BlockSpec((1, V, half), lambda bi: (bi, 0, 0))] * 2,
        out_shape=[jax.ShapeDtypeStruct((bs, V, half), jnp.float32)] * 2,
    )(gp, w, b)


def _kC2_body(gv_ref, v_ref, d_ref, c_ref, sup_ref, fm_ref, gl_ref):
    th = _theta(gv_ref[0], v_ref[0], d_ref[...], 128, 32)
    fm4 = c_ref[0] + jnp.max(th * sup_ref[0], axis=1)
    fm_ref[0] = fm4
    gl_ref[0] = jnp.max(fm4, axis=0, keepdims=True)


def _near_body(v_ref, p1_ref, p2_ref, n1_ref, n2_ref):
    vb = v_ref[0]
    t2 = jnp.sum(vb * vb, axis=1, keepdims=True)

    def nearest(pT):
        g = _bdot(vb, pT)
        s2 = jnp.sum(pT * pT, axis=0, keepdims=True)
        dist = (s2 + t2) - 2.0 * g
        m = jnp.min(dist, axis=1, keepdims=True)
        col = lax.broadcasted_iota(jnp.int32, dist.shape, 1)
        W = dist.shape[1]
        return jnp.min(jnp.where(dist == m, col, W), axis=1)

    n1_ref[0, 0] = nearest(p1_ref[0])
    n2_ref[0, 0] = nearest(p2_ref[0])


def _head_body(fm0_ref, fm1_ref, u23_ref, u4_ref, gl_ref,
               wa0, wa1, wa23, wa4, wag, ba, wb, bb, wc, bc, out_ref):
    x = (_bdot(fm0_ref[0], wa0[...]) + _bdot(fm1_ref[0], wa1[...])
         + _bdot(u23_ref[0], wa23[...]) + _bdot(u4_ref[0], wa4[...])
         + _bdot(gl_ref[0], wag[...]) + ba[...])
    x = jnp.maximum(x, 0.0)
    x = jnp.maximum(_bdot(x, wb[...]) + bb[...], 0.0)
    out_ref[0] = _bdot(x, wc[...]) + bc[...]


def kernel(vertices, d0, w1, b1, d1, w2, b2, d2, w3, b3, d3, w4, b4, d4,
           Wa, ba, Wb, bb, Wc, bc):
    bs, V, _ = vertices.shape
    f32 = jnp.float32
    ba2, bb2, bc2 = ba[None], bb[None], bc[None]
    b1_2, b2_2, b3_2, b4_2 = b1[None], b2[None], b3[None], b4[None]

    vT = jnp.transpose(vertices, (0, 2, 1))
    ni = _knn(vertices, vT, R=512)

    off0 = (jnp.arange(bs, dtype=jnp.int32) * V)[:, None, None]
    idxA = (ni + off0).reshape(-1)

    cpad0 = jnp.pad(vertices, ((0, 0), (0, 0), (0, 125)))
    GV = _sc_gather(cpad0.reshape(bs * V, 128), idxA).reshape(bs, V, 32, 128)

    vp1 = vertices[:, _S1, :]
    vp1T = jnp.transpose(vp1, (0, 2, 1))
    ni2 = _knn(vp1, vp1T, R=512)
    off1 = (jnp.arange(bs, dtype=jnp.int32) * 512)[:, None, None]
    idxB = (ni2 + off1).reshape(-1)
    vp2 = vp1[:, _S2, :]
    vp2T = jnp.transpose(vp2, (0, 2, 1))
    ni3 = _knn(vp2, vp2T, R=128)
    off2 = (jnp.arange(bs, dtype=jnp.int32) * 128)[:, None, None]
    idxC = (ni3 + off2).reshape(-1)

    np1, np2 = pl.pallas_call(
        _near_body,
        grid=(bs, V // 512),
        in_specs=[
            pl.BlockSpec((1, 512, 3), lambda b, r: (b, r, 0)),
            pl.BlockSpec((1, 3, 512), lambda b, r: (b, 0, 0)),
            pl.BlockSpec((1, 3, 128), lambda b, r: (b, 0, 0)),
        ],
        out_specs=[pl.BlockSpec((1, 1, 512),
                                lambda b, r: (b * (V // 512) + r, 0, 0))] * 2,
        out_shape=[jax.ShapeDtypeStruct((bs * (V // 512), 1, 512),
                                        jnp.int32)] * 2,
    )(vertices, vp1T, vp2T)
    np1 = np1.reshape(bs, V)
    np2 = np2.reshape(bs, V)

    cpad1 = jnp.pad(vp1, ((0, 0), (0, 0), (0, 125)))
    GV2 = _sc_gather(cpad1.reshape(bs * 512, 128),
                     idxB).reshape(bs, 512, 32, 128)
    cpad2 = jnp.pad(vp2, ((0, 0), (0, 0), (0, 125)))
    GV3 = _sc_gather(cpad2.reshape(bs * 128, 128),
                     idxC).reshape(bs, 128, 32, 128)

    fm0, c1, s1 = pl.pallas_call(
        functools.partial(_kA_body, 256),
        grid=(bs, V // 256),
        in_specs=[
            pl.BlockSpec((1, 256, 32, 128), lambda b, r: (b, r, 0, 0)),
            pl.BlockSpec((1, 256, 3), lambda b, r: (b, r, 0)),
            pl.BlockSpec((3, 128), lambda b, r: (0, 0)),
            pl.BlockSpec((128, 256), lambda b, r: (0, 0)),
            pl.BlockSpec((1, 256), lambda b, r: (0, 0)),
        ],
        out_specs=[pl.BlockSpec((1, 256, 128), lambda b, r: (b, r, 0))] * 3,
        out_shape=[jax.ShapeDtypeStruct((bs, V, 128), f32)] * 3,
    )(GV, vertices, d0, w1, b1_2)

    g1 = _sc_gather(s1.reshape(bs * V, 128), idxA).reshape(bs, V, 32, 128)
    (fm1,) = _comb(GV, vertices, d1, c1, g1, R=256)

    ip1 = (ni[:, _S1, :4] + off0).reshape(-1)
    gp1 = _sc_gather(fm1.reshape(bs * V, 128), ip1).reshape(bs, 512, 4, 128)

    c2, s2 = _pool_mm(gp1, w2, b2_2)
    g2 = _sc_gather(s2.reshape(bs * 512, 256), idxB).reshape(bs, 512, 32, 256)
    fm2, c3, s3 = _comb(GV2, vp1, d2, c2, g2, R=128, w=w3, b=b3_2)
    g3 = _sc_gather(s3.reshape(bs * 512, 256), idxB).reshape(bs, 512, 32, 256)
    (fm3,) = _comb(GV2, vp1, d3, c3, g3, R=128)

    ip2 = (ni2[:, _S2, :4] + off1).reshape(-1)
    gp2 = _sc_gather(fm3.reshape(bs * 512, 256), ip2).reshape(bs, 128, 4, 256)

    c4, s4 = _pool_mm(gp2, w4, b4_2)
    g4 = _sc_gather(s4.reshape(bs * 128, 512), idxC).reshape(bs, 128, 32, 512)

    fm4, fglob = pl.pallas_call(
        _kC2_body,
        grid=(bs,),
        in_specs=[
            pl.BlockSpec((1, 128, 32, 128), lambda b: (b, 0, 0, 0)),
            pl.BlockSpec((1, 128, 3), lambda b: (b, 0, 0)),
            pl.BlockSpec((3, 512), lambda b: (0, 0)),
            pl.BlockSpec((1, 128, 512), lambda b: (b, 0, 0)),
            pl.BlockSpec((1, 128, 32, 512), lambda b: (b, 0, 0, 0)),
        ],
        out_specs=[pl.BlockSpec((1, 128, 512), lambda b: (b, 0, 0)),
                   pl.BlockSpec((1, 1, 512), lambda b: (b, 0, 0))],
        out_shape=[jax.ShapeDtypeStruct((bs, 128, 512), f32),
                   jax.ShapeDtypeStruct((bs, 1, 512), f32)],
    )(GV3, vp2, d4, c4, g4)

    fm23 = jnp.concatenate([fm2, fm3], axis=-1)
    utab = jnp.concatenate([fm23.reshape(bs * 512, 512),
                            fm4.reshape(bs * 128, 512)], axis=0)
    uidx = jnp.concatenate([(np1 + off1[:, :, 0]).reshape(-1),
                            (np2 + off2[:, :, 0] + bs * 512).reshape(-1)])
    uu = _sc_gather(utab, uidx)
    u23 = uu[:bs * V].reshape(bs, V, 512)
    u4 = uu[bs * V:].reshape(bs, V, 512)

    out = pl.pallas_call(
        _head_body,
        grid=(bs, V // 512),
        in_specs=[
            pl.BlockSpec((1, 512, 128), lambda b, r: (b, r, 0)),
            pl.BlockSpec((1, 512, 128), lambda b, r: (b, r, 0)),
            pl.BlockSpec((1, 512, 512), lambda b, r: (b, r, 0)),
            pl.BlockSpec((1, 512, 512), lambda b, r: (b, r, 0)),
            pl.BlockSpec((1, 1, 512), lambda b, r: (b, 0, 0)),
            pl.BlockSpec((128, 512), lambda b, r: (0, 0)),
            pl.BlockSpec((128, 512), lambda b, r: (0, 0)),
            pl.BlockSpec((512, 512), lambda b, r: (0, 0)),
            pl.BlockSpec((512, 512), lambda b, r: (0, 0)),
            pl.BlockSpec((512, 512), lambda b, r: (0, 0)),
            pl.BlockSpec((1, 512), lambda b, r: (0, 0)),
            pl.BlockSpec((512, 512), lambda b, r: (0, 0)),
            pl.BlockSpec((1, 512), lambda b, r: (0, 0)),
            pl.BlockSpec((512, 50), lambda b, r: (0, 0)),
            pl.BlockSpec((1, 50), lambda b, r: (0, 0)),
        ],
        out_specs=pl.BlockSpec((1, 512, 50), lambda b, r: (b, r, 0)),
        out_shape=jax.ShapeDtypeStruct((bs, V, 50), f32),
    )(fm0, fm1, u23, u4, fglob, Wa[:128], Wa[128:256], Wa[256:768],
      Wa[768:1280], Wa[1280:1792], ba2, Wb, bb2, Wc, bc2)

    return out


# speedup vs baseline: 9.6742x; 1.0025x over previous
import functools

import numpy as np
import jax
import jax.numpy as jnp
from jax import lax
from jax.experimental import pallas as pl
from jax.experimental.pallas import tpu as pltpu
from jax.experimental.pallas import tpu_sc as plsc

_S1 = np.random.RandomState(0).permutation(2048)[:512]
_S2 = np.random.RandomState(1).permutation(512)[:128]

_NW = 32


def _bdot(a, b):
    return jnp.dot(a.astype(jnp.bfloat16), b.astype(jnp.bfloat16),
                   preferred_element_type=jnp.float32)


_NBUF = 4


def _chunk_len(n, D):
    cap = min(128, n, 65536 // (_NBUF * D))
    C = 1
    for c in range(1, cap + 1):
        if n % c == 0:
            C = c
    return C


@functools.lru_cache(maxsize=None)
def _make_sc_gather(T, D, B):
    n = B // _NW
    C = _chunk_len(n, D)
    nchunk = n // C
    nbuf = min(_NBUF, nchunk)
    mesh = plsc.VectorSubcoreMesh(
        core_axis_name="c", subcore_axis_name="s", num_cores=2,
        num_subcores=16)

    @functools.partial(
        pl.kernel,
        out_type=jax.ShapeDtypeStruct((B, D), jnp.float32),
        mesh=mesh,
        scratch_types=(
            [pltpu.VMEM((nchunk, C), jnp.int32)]
            + [pltpu.VMEM((C, D), jnp.float32)] * nbuf
            + [pltpu.SemaphoreType.DMA] * (2 * nbuf)
        ),
    )
    def gather(table_hbm, idx_hbm, out_hbm, idx_v, *bufs):
        rows = bufs[:nbuf]
        gsem = bufs[nbuf:2 * nbuf]
        wsem = bufs[2 * nbuf:]
        wid = lax.axis_index("s") * 2 + lax.axis_index("c")
        row0 = wid * nchunk
        pltpu.sync_copy(idx_hbm.at[pl.ds(row0, nchunk)], idx_v)
        gh = [None] * nbuf
        wh = [None] * nbuf
        for j in range(nchunk):
            b = j % nbuf
            if wh[b] is not None:
                wh[b].wait()
            gh[b] = pltpu.async_copy(table_hbm.at[idx_v.at[j]], rows[b],
                                     gsem[b])
            d = j - (nbuf - 1)
            if d >= 0:
                db = d % nbuf
                gh[db].wait()
                wh[db] = pltpu.async_copy(
                    rows[db], out_hbm.at[pl.ds((row0 + d) * C, C)],
                    wsem[db])
        for d in range(max(0, nchunk - nbuf + 1), nchunk):
            db = d % nbuf
            gh[db].wait()
            wh[db] = pltpu.async_copy(
                rows[db], out_hbm.at[pl.ds((row0 + d) * C, C)], wsem[db])
        for h in wh:
            if h is not None:
                h.wait()

    return gather


def _sc_gather(table, idx):
    T, D = table.shape
    B = idx.shape[0]
    n = B // _NW
    C = _chunk_len(n, D)
    idx2d = idx.reshape(B // C, C)
    return _make_sc_gather(T, D, B)(table, idx2d)


def _knn_body(R, V, K, vb_ref, vT_ref, ni_ref, dist_ref):
    vb = vb_ref[0]
    vT = vT_ref[0]
    g = _bdot(vb, vT)
    q_all = jnp.sum(vT * vT, axis=0, keepdims=True)
    q_row = jnp.sum(vb * vb, axis=1, keepdims=True)
    col = lax.broadcasted_iota(jnp.int32, (R, V), 1)
    dist_ref[...] = (-2.0 * g + q_all) + q_row
    kiota = lax.broadcasted_iota(jnp.int32, (R, K), 1)

    def step(j, ids):
        d = dist_ref[...]
        m = jnp.min(d, axis=1, keepdims=True)
        idx = jnp.min(jnp.where(d == m, col, V), axis=1, keepdims=True)
        dist_ref[...] = jnp.where(col == idx, jnp.inf, d)
        return jnp.where(kiota == j - 1, idx, ids)

    ni_ref[0] = lax.fori_loop(0, K + 1, step, jnp.zeros((R, K), jnp.int32))


def _knn(v, vT, R, K=32):
    bs, V, _ = v.shape
    grid = (bs, V // R)
    return pl.pallas_call(
        functools.partial(_knn_body, R, V, K),
        grid=grid,
        in_specs=[
            pl.BlockSpec((1, R, 3), lambda b, r: (b, r, 0)),
            pl.BlockSpec((1, 3, V), lambda b, r: (b, 0, 0)),
        ],
        out_specs=pl.BlockSpec((1, R, K), lambda b, r: (b, r, 0)),
        out_shape=jax.ShapeDtypeStruct((bs, V, K), jnp.int32),
        scratch_shapes=[pltpu.VMEM((R, V), jnp.float32)],
    )(v, vT)


def _theta(gv, vb, d, R, N):
    n = jnp.sqrt(jnp.sum(d * d, axis=0, keepdims=True))
    sd = d / jnp.maximum(n, 1e-12)
    diff = gv[:, :, :3] - vb[:, None, :]
    nrm = jnp.sqrt(jnp.sum(diff * diff, axis=2, keepdims=True))
    nd = diff / jnp.maximum(nrm, 1e-12)
    C = d.shape[1]
    return jnp.maximum(_bdot(nd.reshape(R * N, 3), sd), 0.0) \
        .reshape(R, N, C)


def _kA_body(R, gv_ref, v_ref, d_ref, w_ref, b_ref, fm0_ref, c1_ref, s1_ref):
    th = _theta(gv_ref[0], v_ref[0], d_ref[...], R, 32)
    fm0 = jnp.maximum(jnp.max(th, axis=1), 0.0)
    fout = _bdot(fm0, w_ref[...]) + b_ref[...]
    fm0_ref[0] = fm0
    c1_ref[0] = fout[:, :128]
    s1_ref[0] = fout[:, 128:]


def _comb_body(R, relu, w_next, *refs):
    if w_next:
        (gv_ref, v_ref, d_ref, c_ref, sup_ref, w_ref, b_ref,
         fm_ref, cn_ref, sn_ref) = refs
    else:
        gv_ref, v_ref, d_ref, c_ref, sup_ref, fm_ref = refs
    th = _theta(gv_ref[0], v_ref[0], d_ref[...], R, 32)
    out = c_ref[0] + jnp.max(th * sup_ref[0], axis=1)
    if relu:
        out = jnp.maximum(out, 0.0)
    fm_ref[0] = out
    if w_next:
        fout = _bdot(out, w_ref[...]) + b_ref[...]
        half = w_ref.shape[1] // 2
        cn_ref[0] = fout[:, :half]
        sn_ref[0] = fout[:, half:]


def _comb(GV, v, d, c, sup, R, relu=True, w=None, b=None):
    bs, V, N, C = sup.shape
    grid = (bs, V // R)
    in_arrs = [GV, v, d, c, sup]
    in_specs = [
        pl.BlockSpec((1, R, N, 128), lambda bi, r: (bi, r, 0, 0)),
        pl.BlockSpec((1, R, 3), lambda bi, r: (bi, r, 0)),
        pl.BlockSpec(d.shape, lambda bi, r: (0, 0)),
        pl.BlockSpec((1, R, C), lambda bi, r: (bi, r, 0)),
        pl.BlockSpec((1, R, N, C), lambda bi, r: (bi, r, 0, 0)),
    ]
    out_shapes = [jax.ShapeDtypeStruct((bs, V, C), jnp.float32)]
    out_specs = [pl.BlockSpec((1, R, C), lambda bi, r: (bi, r, 0))]
    if w is not None:
        in_arrs += [w, b]
        in_specs += [pl.BlockSpec(w.shape, lambda bi, r: (0, 0)),
                     pl.BlockSpec(b.shape, lambda bi, r: (0, 0))]
        half = w.shape[1] // 2
        out_shapes += [jax.ShapeDtypeStruct((bs, V, half), jnp.float32)] * 2
        out_specs += [pl.BlockSpec((1, R, half), lambda bi, r: (bi, r, 0))] * 2
    return pl.pallas_call(
        functools.partial(_comb_body, R, relu, w is not None),
        grid=grid, in_specs=in_specs, out_specs=out_specs,
        out_shape=out_shapes,
    )(*in_arrs)


def _pool_mm_body(g_ref, w_ref, b_ref, c_ref, s_ref):
    fmp = jnp.max(g_ref[0], axis=1)
    fout = _bdot(fmp, w_ref[...]) + b_ref[...]
    half = w_ref.shape[1] // 2
    c_ref[0] = fout[:, :half]
    s_ref[0] = fout[:, half:]


def _pool_mm(gp, w, b):
    bs, V, pn, Cin = gp.shape
    half = w.shape[1] // 2
    return pl.pallas_call(
        _pool_mm_body,
        grid=(bs,),
        in_specs=[
            pl.BlockSpec((1, V, pn, Cin), lambda bi: (bi, 0, 0, 0)),
            pl.BlockSpec(w.shape, lambda bi: (0, 0)),
            pl.BlockSpec(b.shape, lambda bi: (0, 0)),
        ],
        out_specs=[pl.BlockSpec((1, V, half), lambda bi: (bi, 0, 0))] * 2,
        out_shape=[jax.ShapeDtypeStruct((bs, V, half), jnp.float32)] * 2,
    )(gp, w, b)


def _kC2_body(gv_ref, v_ref, d_ref, c_ref, sup_ref, fm_ref, gl_ref):
    th = _theta(gv_ref[0], v_ref[0], d_ref[...], 128, 32)
    fm4 = c_ref[0] + jnp.max(th * sup_ref[0], axis=1)
    fm_ref[0] = fm4
    gl_ref[0] = jnp.max(fm4, axis=0, keepdims=True)


def _near_body(v_ref, p1_ref, p2_ref, n1_ref, n2_ref):
    vb = v_ref[0]
    t2 = jnp.sum(vb * vb, axis=1, keepdims=True)

    def nearest(pT):
        g = _bdot(vb, pT)
        s2 = jnp.sum(pT * pT, axis=0, keepdims=True)
        dist = (s2 + t2) - 2.0 * g
        m = jnp.min(dist, axis=1, keepdims=True)
        col = lax.broadcasted_iota(jnp.int32, dist.shape, 1)
        W = dist.shape[1]
        return jnp.min(jnp.where(dist == m, col, W), axis=1)

    n1_ref[0, 0] = nearest(p1_ref[0])
    n2_ref[0, 0] = nearest(p2_ref[0])


def _head_body(fm0_ref, fm1_ref, u23_ref, u4_ref, gl_ref,
               wa0, wa1, wa23, wa4, wag, ba, wb, bb, wc, bc, out_ref):
    x = (_bdot(fm0_ref[0], wa0[...]) + _bdot(fm1_ref[0], wa1[...])
         + _bdot(u23_ref[0], wa23[...]) + _bdot(u4_ref[0], wa4[...])
         + _bdot(gl_ref[0], wag[...]) + ba[...])
    x = jnp.maximum(x, 0.0)
    x = jnp.maximum(_bdot(x, wb[...]) + bb[...], 0.0)
    out_ref[0] = _bdot(x, wc[...]) + bc[...]


def kernel(vertices, d0, w1, b1, d1, w2, b2, d2, w3, b3, d3, w4, b4, d4,
           Wa, ba, Wb, bb, Wc, bc):
    bs, V, _ = vertices.shape
    f32 = jnp.float32
    ba2, bb2, bc2 = ba[None], bb[None], bc[None]
    b1_2, b2_2, b3_2, b4_2 = b1[None], b2[None], b3[None], b4[None]

    vT = jnp.transpose(vertices, (0, 2, 1))
    ni = _knn(vertices, vT, R=512)

    off0 = (jnp.arange(bs, dtype=jnp.int32) * V)[:, None, None]
    idxA = (ni + off0).reshape(-1)

    cpad0 = jnp.pad(vertices, ((0, 0), (0, 0), (0, 125)))
    GV = _sc_gather(cpad0.reshape(bs * V, 128), idxA).reshape(bs, V, 32, 128)

    vp1 = vertices[:, _S1, :]
    vp1T = jnp.transpose(vp1, (0, 2, 1))
    ni2 = _knn(vp1, vp1T, R=512)
    off1 = (jnp.arange(bs, dtype=jnp.int32) * 512)[:, None, None]
    idxB = (ni2 + off1).reshape(-1)
    vp2 = vp1[:, _S2, :]
    vp2T = jnp.transpose(vp2, (0, 2, 1))
    ni3 = _knn(vp2, vp2T, R=128)
    off2 = (jnp.arange(bs, dtype=jnp.int32) * 128)[:, None, None]
    idxC = (ni3 + off2).reshape(-1)

    np1, np2 = pl.pallas_call(
        _near_body,
        grid=(bs, V // 512),
        in_specs=[
            pl.BlockSpec((1, 512, 3), lambda b, r: (b, r, 0)),
            pl.BlockSpec((1, 3, 512), lambda b, r: (b, 0, 0)),
            pl.BlockSpec((1, 3, 128), lambda b, r: (b, 0, 0)),
        ],
        out_specs=[pl.BlockSpec((1, 1, 512),
                                lambda b, r: (b * (V // 512) + r, 0, 0))] * 2,
        out_shape=[jax.ShapeDtypeStruct((bs * (V // 512), 1, 512),
                                        jnp.int32)] * 2,
    )(vertices, vp1T, vp2T)
    np1 = np1.reshape(bs, V)
    np2 = np2.reshape(bs, V)

    cpad1 = jnp.pad(vp1, ((0, 0), (0, 0), (0, 125)))
    GV2 = _sc_gather(cpad1.reshape(bs * 512, 128),
                     idxB).reshape(bs, 512, 32, 128)
    cpad2 = jnp.pad(vp2, ((0, 0), (0, 0), (0, 125)))
    GV3 = _sc_gather(cpad2.reshape(bs * 128, 128),
                     idxC).reshape(bs, 128, 32, 128)

    fm0, c1, s1 = pl.pallas_call(
        functools.partial(_kA_body, 256),
        grid=(bs, V // 256),
        in_specs=[
            pl.BlockSpec((1, 256, 32, 128), lambda b, r: (b, r, 0, 0)),
            pl.BlockSpec((1, 256, 3), lambda b, r: (b, r, 0)),
            pl.BlockSpec((3, 128), lambda b, r: (0, 0)),
            pl.BlockSpec((128, 256), lambda b, r: (0, 0)),
            pl.BlockSpec((1, 256), lambda b, r: (0, 0)),
        ],
        out_specs=[pl.BlockSpec((1, 256, 128), lambda b, r: (b, r, 0))] * 3,
        out_shape=[jax.ShapeDtypeStruct((bs, V, 128), f32)] * 3,
    )(GV, vertices, d0, w1, b1_2)

    g1 = _sc_gather(s1.reshape(bs * V, 128), idxA).reshape(bs, V, 32, 128)
    (fm1,) = _comb(GV, vertices, d1, c1, g1, R=256)

    ip1 = (ni[:, _S1, :4] + off0).reshape(-1)
    gp1 = _sc_gather(fm1.reshape(bs * V, 128), ip1).reshape(bs, 512, 4, 128)

    c2, s2 = _pool_mm(gp1, w2, b2_2)
    g2 = _sc_gather(s2.reshape(bs * 512, 256), idxB).reshape(bs, 512, 32, 256)
    fm2, c3, s3 = _comb(GV2, vp1, d2, c2, g2, R=128, w=w3, b=b3_2)
    g3 = _sc_gather(s3.reshape(bs * 512, 256), idxB).reshape(bs, 512, 32, 256)
    (fm3,) = _comb(GV2, vp1, d3, c3, g3, R=128)

    ip2 = (ni2[:, _S2, :4] + off1).reshape(-1)
    gp2 = _sc_gather(fm3.reshape(bs * 512, 256), ip2).reshape(bs, 128, 4, 256)

    c4, s4 = _pool_mm(gp2, w4, b4_2)
    g4 = _sc_gather(s4.reshape(bs * 128, 512), idxC).reshape(bs, 128, 32, 512)

    fm4, fglob = pl.pallas_call(
        _kC2_body,
        grid=(bs,),
        in_specs=[
            pl.BlockSpec((1, 128, 32, 128), lambda b: (b, 0, 0, 0)),
            pl.BlockSpec((1, 128, 3), lambda b: (b, 0, 0)),
            pl.BlockSpec((3, 512), lambda b: (0, 0)),
            pl.BlockSpec((1, 128, 512), lambda b: (b, 0, 0)),
            pl.BlockSpec((1, 128, 32, 512), lambda b: (b, 0, 0, 0)),
        ],
        out_specs=[pl.BlockSpec((1, 128, 512), lambda b: (b, 0, 0)),
                   pl.BlockSpec((1, 1, 512), lambda b: (b, 0, 0))],
        out_shape=[jax.ShapeDtypeStruct((bs, 128, 512), f32),
                   jax.ShapeDtypeStruct((bs, 1, 512), f32)],
    )(GV3, vp2, d4, c4, g4)

    fm23 = jnp.concatenate([fm2, fm3], axis=-1)
    utab = jnp.concatenate([fm23.reshape(bs * 512, 512),
                            fm4.reshape(bs * 128, 512)], axis=0)
    uidx = jnp.concatenate([(np1 + off1[:, :, 0]).reshape(-1),
                            (np2 + off2[:, :, 0] + bs * 512).reshape(-1)])
    uu = _sc_gather(utab, uidx)
    u23 = uu[:bs * V].reshape(bs, V, 512)
    u4 = uu[bs * V:].reshape(bs, V, 512)

    out = pl.pallas_call(
        _head_body,
        grid=(bs, V // 512),
        in_specs=[
            pl.BlockSpec((1, 512, 128), lambda b, r: (b, r, 0)),
            pl.BlockSpec((1, 512, 128), lambda b, r: (b, r, 0)),
            pl.BlockSpec((1, 512, 512), lambda b, r: (b, r, 0)),
            pl.BlockSpec((1, 512, 512), lambda b, r: (b, r, 0)),
            pl.BlockSpec((1, 1, 512), lambda b, r: (b, 0, 0)),
            pl.BlockSpec((128, 512), lambda b, r: (0, 0)),
            pl.BlockSpec((128, 512), lambda b, r: (0, 0)),
            pl.BlockSpec((512, 512), lambda b, r: (0, 0)),
            pl.BlockSpec((512, 512), lambda b, r: (0, 0)),
            pl.BlockSpec((512, 512), lambda b, r: (0, 0)),
            pl.BlockSpec((1, 512), lambda b, r: (0, 0)),
            pl.BlockSpec((512, 512), lambda b, r: (0, 0)),
            pl.BlockSpec((1, 512), lambda b, r: (0, 0)),
            pl.BlockSpec((512, 50), lambda b, r: (0, 0)),
            pl.BlockSpec((1, 50), lambda b, r: (0, 0)),
        ],
        out_specs=pl.BlockSpec((1, 512, 50), lambda b, r: (b, r, 0)),
        out_shape=jax.ShapeDtypeStruct((bs, V, 50), f32),
    )(fm0, fm1, u23, u4, fglob, Wa[:128], Wa[128:256], Wa[256:768],
      Wa[768:1280], Wa[1280:1792], ba2, Wb, bb2, Wc, bc2)

    return out
